# baseline (speedup 1.0000x reference)
import functools

import jax
import jax.numpy as jnp
from jax import lax
from jax.experimental import pallas as pl
from jax.experimental.pallas import tpu as pltpu
from jax.experimental.pallas import tpu_sc as plsc

N_NODES = 10000
D = 128
N_EDGES = 320000

_NC = 2
_NS = 16
_NW = _NC * _NS
_L = 16
_ROW_BLK = 2000

_CHUNK = 128
_BASE_CHUNKS = 78
_MAX_EDGES = (_BASE_CHUNKS + 1) * _CHUNK


def _matvec_body(x_ref, wrel_ref, wroot_ref, b_ref, srel_ref, base_ref):
    xb = x_ref[...]
    dn = (((1,), (1,)), ((), ()))
    srel_ref[...] = jax.lax.dot_general(
        wrel_ref[...], xb, dn, preferred_element_type=jnp.float32
    )
    base_ref[...] = (
        jax.lax.dot_general(wroot_ref[...], xb, dn, preferred_element_type=jnp.float32)
        + b_ref[0, 0]
    )


def _edge_body(zeros_hbm, srel_hbm, edge_hbm, out_hbm, srel_v, edges_v, acc_v, sem):
    wid = lax.axis_index("s") * _NC + lax.axis_index("c")
    extra = jnp.maximum(wid - 28, 0)
    c0 = pl.multiple_of((wid * _BASE_CHUNKS + extra) * _CHUNK, _CHUNK)
    nvec = (_BASE_CHUNKS * _CHUNK) // _L + jnp.where(wid >= 28, 8, 0)

    cps = [
        pltpu.async_copy(zeros_hbm, acc_v, sem),
        pltpu.async_copy(srel_hbm.at[0], srel_v, sem),
        pltpu.async_copy(edge_hbm.at[:, pl.ds(c0, _MAX_EDGES)], edges_v, sem),
    ]
    for cp in cps:
        cp.wait()

    @plsc.parallel_loop(0, nvec, unroll=4)
    def edge_step(k):
        sl = pl.ds(k * _L, _L)
        vals = plsc.load_gather(srel_v, [edges_v[0, sl]])
        plsc.addupdate_scatter(acc_v, [edges_v[1, sl]], vals)

    pltpu.sync_copy(acc_v, out_hbm.at[wid])


_edge_kernel = functools.partial(
    pl.kernel,
    mesh=plsc.VectorSubcoreMesh(core_axis_name="c", subcore_axis_name="s"),
    compiler_params=pltpu.CompilerParams(needs_layout_passes=False),
    out_type=jax.ShapeDtypeStruct((_NW, N_NODES), jnp.float32),
    scratch_types=[
        pltpu.VMEM((N_NODES,), jnp.float32),
        pltpu.VMEM((2, _MAX_EDGES), jnp.int32),
        pltpu.VMEM((N_NODES,), jnp.float32),
        pltpu.SemaphoreType.DMA,
    ],
)(_edge_body)


def _combine_body(p_ref, base_ref, out_ref):
    out_ref[...] = jnp.sum(p_ref[...], axis=0, keepdims=True) + base_ref[...]


def kernel(x, edge_index, W_rel, b_rel, W_root):
    edges = edge_index.astype(jnp.int32)
    srel, base = pl.pallas_call(
        _matvec_body,
        out_shape=[
            jax.ShapeDtypeStruct((1, N_NODES), jnp.float32),
            jax.ShapeDtypeStruct((1, N_NODES), jnp.float32),
        ],
    )(x, W_rel.reshape(1, D), W_root.reshape(1, D), b_rel.reshape(1, 1))

    partials = _edge_kernel(jnp.zeros((N_NODES,), jnp.float32), srel, edges)

    out_row = pl.pallas_call(
        _combine_body,
        out_shape=jax.ShapeDtypeStruct((1, N_NODES), jnp.float32),
    )(partials, base)
    return out_row.reshape(N_NODES, 1)


# speedup vs baseline: 1.0800x; 1.0800x over previous
import functools

import jax
import jax.numpy as jnp
from jax import lax
from jax.experimental import pallas as pl
from jax.experimental.pallas import tpu as pltpu
from jax.experimental.pallas import tpu_sc as plsc

N_NODES = 10000
D = 128
N_EDGES = 320000

_NC = 2
_NS = 16
_NW = _NC * _NS
_L = 16
_ROW_BLK = 2000

_CHUNK = 128
_BASE_CHUNKS = 78
_MAX_EDGES = (_BASE_CHUNKS + 1) * _CHUNK


def _matvec_body(x_ref, wrel_ref, wroot_ref, b_ref, srel_ref, base_ref):
    xb = x_ref[...]
    dn = (((1,), (1,)), ((), ()))
    srel_ref[...] = jax.lax.dot_general(
        wrel_ref[...], xb, dn, preferred_element_type=jnp.float32
    )
    base_ref[...] = (
        jax.lax.dot_general(wroot_ref[...], xb, dn, preferred_element_type=jnp.float32)
        + b_ref[0, 0]
    )


def _edge_body(srel_hbm, edge_hbm, out_hbm, srel_v, edges_v, acc_v, sem):
    wid = lax.axis_index("s") * _NC + lax.axis_index("c")
    extra = jnp.maximum(wid - 28, 0)
    c0 = pl.multiple_of((wid * _BASE_CHUNKS + extra) * _CHUNK, _CHUNK)
    nvec = (_BASE_CHUNKS * _CHUNK) // _L + jnp.where(wid >= 28, 8, 0)

    cps = [
        pltpu.async_copy(srel_hbm.at[0], srel_v, sem),
        pltpu.async_copy(edge_hbm.at[:, pl.ds(c0, _MAX_EDGES)], edges_v, sem),
    ]

    zero16 = jnp.zeros((_L,), jnp.float32)

    @plsc.parallel_loop(0, N_NODES // _L, unroll=8)
    def zero_step(i):
        acc_v[pl.ds(i * _L, _L)] = zero16

    for cp in cps:
        cp.wait()

    @plsc.parallel_loop(0, nvec, unroll=8)
    def edge_step(k):
        sl = pl.ds(k * _L, _L)
        vals = plsc.load_gather(srel_v, [edges_v[0, sl]])
        plsc.addupdate_scatter(acc_v, [edges_v[1, sl]], vals)

    pltpu.sync_copy(acc_v, out_hbm.at[wid])


_edge_kernel = functools.partial(
    pl.kernel,
    mesh=plsc.VectorSubcoreMesh(core_axis_name="c", subcore_axis_name="s"),
    compiler_params=pltpu.CompilerParams(needs_layout_passes=False),
    out_type=jax.ShapeDtypeStruct((_NW, N_NODES), jnp.float32),
    scratch_types=[
        pltpu.VMEM((N_NODES,), jnp.float32),
        pltpu.VMEM((2, _MAX_EDGES), jnp.int32),
        pltpu.VMEM((N_NODES,), jnp.float32),
        pltpu.SemaphoreType.DMA,
    ],
)(_edge_body)


def _combine_body(p_ref, base_ref, out_ref):
    out_ref[...] = jnp.sum(p_ref[...], axis=0, keepdims=True) + base_ref[...]


def kernel(x, edge_index, W_rel, b_rel, W_root):
    edges = edge_index.astype(jnp.int32)
    srel, base = pl.pallas_call(
        _matvec_body,
        out_shape=[
            jax.ShapeDtypeStruct((1, N_NODES), jnp.float32),
            jax.ShapeDtypeStruct((1, N_NODES), jnp.float32),
        ],
    )(x, W_rel.reshape(1, D), W_root.reshape(1, D), b_rel.reshape(1, 1))

    partials = _edge_kernel(srel, edges)

    out_row = pl.pallas_call(
        _combine_body,
        out_shape=jax.ShapeDtypeStruct((1, N_NODES), jnp.float32),
    )(partials, base)
    return out_row.reshape(N_NODES, 1)
